# baseline (speedup 1.0000x reference)
import jax
import jax.numpy as jnp
from jax import lax
from jax.experimental import pallas as pl
from jax.experimental.pallas import tpu as pltpu
from jax.experimental.pallas import tpu_sc as plsc

B = 4096
L = 200
EMB = 64
N = B * L
NW = 32
NB_W = B // NW
SEG0 = 104
SEG1 = 96
SCALE = 8.0


def _body(tokens_hbm, table_hbm, out_hbm, idx_v, rows_v, gsem):
    wid = lax.axis_index("s") * 2 + lax.axis_index("c")
    b0 = wid * NB_W

    def brow(i, carry):
        b = b0 + i
        base = b * L
        for l0, seg in ((0, SEG0), (SEG0, SEG1)):
            idx = idx_v.at[pl.ds(0, seg)]
            rows = rows_v.at[pl.ds(0, seg)]
            pltpu.sync_copy(tokens_hbm.at[pl.ds(base + l0, seg)], idx)
            pltpu.async_copy(table_hbm.at[idx], rows, gsem).wait()

            def row(r, c2):
                for j in range(EMB // 16):
                    rows_v[r, pl.ds(16 * j, 16)] = (
                        rows_v[r, pl.ds(16 * j, 16)] * SCALE)
                return c2

            lax.fori_loop(0, seg, row, 0)
            pltpu.sync_copy(rows, out_hbm.at[b, pl.ds(l0, seg), :])
        return carry

    lax.fori_loop(0, NB_W, brow, 0)


def kernel(tokens, table):
    flat = tokens.reshape(N).astype(jnp.int32)
    mesh = plsc.VectorSubcoreMesh(core_axis_name="c", subcore_axis_name="s")
    out = pl.kernel(
        _body,
        out_type=jax.ShapeDtypeStruct((B, L, EMB), jnp.float32),
        mesh=mesh,
        scratch_types=[
            pltpu.VMEM((SEG0,), jnp.int32),
            pltpu.VMEM((SEG0, EMB), jnp.float32),
            pltpu.SemaphoreType.DMA,
        ],
        compiler_params=pltpu.CompilerParams(use_tc_tiling_on_sc=False),
    )(flat, table)
    return out
